# baseline (speedup 1.0000x reference)
import functools

import jax
import jax.numpy as jnp
from jax.experimental import pallas as pl

N, F, H1, H2, C = 10000, 256, 256, 256, 64

BM = 400


def _small_matmul_kernel(x_ref, w_ref, o_ref):
    a = x_ref[...].astype(jnp.bfloat16)
    b = w_ref[...].astype(jnp.bfloat16)
    o_ref[...] = jnp.dot(a, b, preferred_element_type=jnp.float32).astype(
        jnp.bfloat16
    )


def _small_matmul(x, w, bm=1000):
    m, k = x.shape
    _, n = w.shape
    return pl.pallas_call(
        _small_matmul_kernel,
        grid=(m // bm,),
        in_specs=[
            pl.BlockSpec((bm, k), lambda i: (i, 0)),
            pl.BlockSpec((k, n), lambda i: (0, 0)),
        ],
        out_specs=pl.BlockSpec((bm, n), lambda i: (i, 0)),
        out_shape=jax.ShapeDtypeStruct((m, n), jnp.bfloat16),
    )(x, w)


def _layer_mid_kernel(adj_ref, sup_ref, b_ref, w_next_ref, o_ref):
    a = adj_ref[...].astype(jnp.bfloat16)
    h = jnp.dot(a, sup_ref[...], preferred_element_type=jnp.float32)
    h = jnp.maximum(h + b_ref[...], 0.0)
    o_ref[...] = jnp.dot(
        h.astype(jnp.bfloat16), w_next_ref[...], preferred_element_type=jnp.float32
    ).astype(jnp.bfloat16)


def _layer_mid(adj, support, b, w_next):
    n_out = w_next.shape[1]
    return pl.pallas_call(
        _layer_mid_kernel,
        grid=(N // BM,),
        in_specs=[
            pl.BlockSpec((BM, N), lambda i: (i, 0)),
            pl.BlockSpec((N, H1), lambda i: (0, 0)),
            pl.BlockSpec((1, H1), lambda i: (0, 0)),
            pl.BlockSpec((H1, n_out), lambda i: (0, 0)),
        ],
        out_specs=pl.BlockSpec((BM, n_out), lambda i: (i, 0)),
        out_shape=jax.ShapeDtypeStruct((N, n_out), jnp.bfloat16),
    )(adj, support, b.reshape(1, -1), w_next)


def _layer_final_kernel(adj_ref, sup_ref, b_ref, wf_ref, bf_ref, o_ref):
    a = adj_ref[...].astype(jnp.bfloat16)
    h = jnp.dot(a, sup_ref[...], preferred_element_type=jnp.float32)
    h = jnp.maximum(h + b_ref[...], 0.0)
    logits = (
        jnp.dot(h.astype(jnp.bfloat16), wf_ref[...], preferred_element_type=jnp.float32)
        + bf_ref[...]
    )
    m = jnp.max(logits, axis=1, keepdims=True)
    s = logits - m
    lse = jnp.log(jnp.sum(jnp.exp(s), axis=1, keepdims=True))
    o_ref[...] = s - lse


def _layer_final(adj, support, b, wf, bfin):
    return pl.pallas_call(
        _layer_final_kernel,
        grid=(N // BM,),
        in_specs=[
            pl.BlockSpec((BM, N), lambda i: (i, 0)),
            pl.BlockSpec((N, H2), lambda i: (0, 0)),
            pl.BlockSpec((1, H2), lambda i: (0, 0)),
            pl.BlockSpec((H2, C), lambda i: (0, 0)),
            pl.BlockSpec((1, C), lambda i: (0, 0)),
        ],
        out_specs=pl.BlockSpec((BM, C), lambda i: (i, 0)),
        out_shape=jax.ShapeDtypeStruct((N, C), jnp.float32),
    )(adj, support, b.reshape(1, -1), wf, bfin.reshape(1, -1))


@jax.jit
def kernel(x, adj, W1, b1, W2, b2, Wf, bf):
    support1 = _small_matmul(x, W1)
    support2 = _layer_mid(adj, support1, b1, W2)
    return _layer_final(adj, support2, b2, Wf, bf)
